# baseline (speedup 1.0000x reference)
import jax
import jax.numpy as jnp
from jax import lax
from jax.experimental import pallas as pl
from jax.experimental.pallas import tpu as pltpu
from jax.experimental.pallas import tpu_sc as plsc

_VOCAB = 1000000
_NUM_CODES = 256
_EMBED_DIM = 64
_BATCH = 4096
_FIELDS = 26

_TOTAL = _BATCH * _FIELDS
_NW = 32
_PER_W = _TOTAL // _NW
_CHUNK = 128
_NCH = _PER_W // _CHUNK

_BB = 512
_GRID = _BATCH // _BB


def _sc_codes_body(ids_hbm, bits_hbm, codes_hbm, idx_v, codes_v, csem):
    wid = lax.axis_index("s") * 2 + lax.axis_index("c")

    pltpu.sync_copy(ids_hbm.at[wid], idx_v)

    def fire(j, carry):
        pltpu.async_copy(bits_hbm.at[idx_v.at[j]], codes_v.at[j], csem)
        return carry
    lax.fori_loop(0, _NCH, fire, None)
    pltpu.make_async_copy(ids_hbm.at[wid], codes_v, csem).wait()

    pltpu.sync_copy(codes_v, codes_hbm.at[pl.ds(wid * _NCH, _NCH)])


_RPB = _BB // _CHUNK


def _tc_decode_body(codes_ref, cbth_ref, cbtl_ref, out_ref):
    i = pl.program_id(0)
    cls = lax.broadcasted_iota(jnp.int32, (_NUM_CODES, _BB), 0)
    dn = (((1,), (0,)), ((), ()))
    for f in range(_FIELDS):
        r0 = f * (_BATCH // _CHUNK) + _RPB * i
        row = jnp.concatenate(
            [codes_ref[pl.ds(r0 + q, 1), :] for q in range(_RPB)], axis=1)
        bc = jnp.broadcast_to(row, (_NUM_CODES, _BB))
        oht = (bc == cls).astype(jnp.bfloat16)
        acc = lax.dot_general(cbth_ref[...], oht, dn,
                              preferred_element_type=jnp.float32)
        acc = acc + lax.dot_general(cbtl_ref[...], oht, dn,
                                    preferred_element_type=jnp.float32)
        out_ref[f] = acc


@jax.jit
def kernel(input, bit_arr, codebook):
    ids = input.T.reshape(_NW, _NCH, _CHUNK)

    mesh = plsc.VectorSubcoreMesh(core_axis_name="c", subcore_axis_name="s")
    sc_codes = pl.kernel(
        _sc_codes_body,
        out_type=jax.ShapeDtypeStruct((_NW * _NCH, _CHUNK), jnp.int32),
        mesh=mesh,
        scratch_types=[
            pltpu.VMEM((_NCH, _CHUNK), jnp.int32),
            pltpu.VMEM((_NCH, _CHUNK), jnp.int32),
            pltpu.SemaphoreType.DMA,
        ],
        compiler_params=pltpu.CompilerParams(use_tc_tiling_on_sc=False),
    )
    codes = sc_codes(ids, bit_arr)

    cb_hi = codebook.astype(jnp.bfloat16)
    cb_lo = (codebook - cb_hi.astype(jnp.float32)).astype(jnp.bfloat16)

    decode = pl.pallas_call(
        _tc_decode_body,
        grid=(_GRID,),
        in_specs=[
            pl.BlockSpec((_NW * _NCH, _CHUNK), lambda i: (0, 0)),
            pl.BlockSpec((_EMBED_DIM, _NUM_CODES), lambda i: (0, 0)),
            pl.BlockSpec((_EMBED_DIM, _NUM_CODES), lambda i: (0, 0)),
        ],
        out_specs=pl.BlockSpec((_FIELDS, _EMBED_DIM, _BB), lambda i: (0, 0, i)),
        out_shape=jax.ShapeDtypeStruct((_FIELDS, _EMBED_DIM, _BATCH),
                                       jnp.float32),
    )
    out_t = decode(codes, cb_hi.T, cb_lo.T)
    return jnp.transpose(out_t, (2, 0, 1))


# speedup vs baseline: 1.2091x; 1.2091x over previous
import jax
import jax.numpy as jnp
from jax import lax
from jax.experimental import pallas as pl
from jax.experimental.pallas import tpu as pltpu
from jax.experimental.pallas import tpu_sc as plsc

_VOCAB = 1000000
_NUM_CODES = 256
_EMBED_DIM = 64
_BATCH = 4096
_FIELDS = 26

_TOTAL = _BATCH * _FIELDS
_NW = 32
_PER_W = _TOTAL // _NW
_CHUNK = 128
_NCH = _PER_W // _CHUNK

_BB = 512
_GRID = _BATCH // _BB


def _sc_codes_body(ids_hbm, bits_hbm, codes_hbm, idx_v, codes_v, csem):
    wid = lax.axis_index("s") * 2 + lax.axis_index("c")

    pltpu.sync_copy(ids_hbm.at[wid], idx_v)

    def fire(j, carry):
        pltpu.async_copy(bits_hbm.at[idx_v.at[j]], codes_v.at[j], csem)
        return carry
    lax.fori_loop(0, _NCH, fire, None)
    pltpu.make_async_copy(ids_hbm.at[wid], codes_v, csem).wait()

    pltpu.sync_copy(codes_v, codes_hbm.at[pl.ds(wid * _NCH, _NCH)])


_RPB = _BB // _CHUNK


def _tc_decode_body(codes_ref, cbt_ref, out_ref):
    i = pl.program_id(0)
    cls = lax.broadcasted_iota(jnp.int32, (_NUM_CODES, _BB), 0)
    dn = (((1,), (0,)), ((), ()))
    for f in range(_FIELDS):
        r0 = f * (_BATCH // _CHUNK) + _RPB * i
        row = jnp.concatenate(
            [codes_ref[pl.ds(r0 + q, 1), :] for q in range(_RPB)], axis=1)
        bc = jnp.broadcast_to(row, (_NUM_CODES, _BB))
        oht = (bc == cls).astype(jnp.bfloat16)
        acc = lax.dot_general(cbt_ref[...], oht, dn,
                              preferred_element_type=jnp.float32)
        out_ref[f] = acc


@jax.jit
def kernel(input, bit_arr, codebook):
    ids = input.T.reshape(_NW, _NCH, _CHUNK)

    mesh = plsc.VectorSubcoreMesh(core_axis_name="c", subcore_axis_name="s")
    sc_codes = pl.kernel(
        _sc_codes_body,
        out_type=jax.ShapeDtypeStruct((_NW * _NCH, _CHUNK), jnp.int32),
        mesh=mesh,
        scratch_types=[
            pltpu.VMEM((_NCH, _CHUNK), jnp.int32),
            pltpu.VMEM((_NCH, _CHUNK), jnp.int32),
            pltpu.SemaphoreType.DMA,
        ],
        compiler_params=pltpu.CompilerParams(use_tc_tiling_on_sc=False),
    )
    codes = sc_codes(ids, bit_arr)

    cb_hi = codebook.astype(jnp.bfloat16)

    decode = pl.pallas_call(
        _tc_decode_body,
        grid=(_GRID,),
        in_specs=[
            pl.BlockSpec((_NW * _NCH, _CHUNK), lambda i: (0, 0)),
            pl.BlockSpec((_EMBED_DIM, _NUM_CODES), lambda i: (0, 0)),
        ],
        out_specs=pl.BlockSpec((_FIELDS, _EMBED_DIM, _BB), lambda i: (0, 0, i)),
        out_shape=jax.ShapeDtypeStruct((_FIELDS, _EMBED_DIM, _BATCH),
                                       jnp.float32),
    )
    out_t = decode(codes, cb_hi.T)
    return jnp.transpose(out_t, (2, 0, 1))


# speedup vs baseline: 1.2204x; 1.0094x over previous
import jax
import jax.numpy as jnp
from jax import lax
from jax.experimental import pallas as pl
from jax.experimental.pallas import tpu as pltpu
from jax.experimental.pallas import tpu_sc as plsc

_VOCAB = 1000000
_NUM_CODES = 256
_EMBED_DIM = 64
_BATCH = 4096
_FIELDS = 26

_TOTAL = _BATCH * _FIELDS
_NW = 32
_PER_W = _TOTAL // _NW
_CHUNK = 128
_NCH = _PER_W // _CHUNK

_BB = 1024
_GRID = _BATCH // _BB


def _sc_codes_body(ids_hbm, bits_hbm, codes_hbm, idx_v, codes_v, csem):
    wid = lax.axis_index("s") * 2 + lax.axis_index("c")

    pltpu.sync_copy(ids_hbm.at[wid], idx_v)

    def fire(j, carry):
        pltpu.async_copy(bits_hbm.at[idx_v.at[j]], codes_v.at[j], csem)
        return carry
    lax.fori_loop(0, _NCH, fire, None)
    pltpu.make_async_copy(ids_hbm.at[wid], codes_v, csem).wait()

    pltpu.sync_copy(codes_v, codes_hbm.at[pl.ds(wid * _NCH, _NCH)])


_RPB = _BB // _CHUNK


def _tc_decode_body(codes_ref, cbt_ref, out_ref):
    i = pl.program_id(0)
    cls = lax.broadcasted_iota(jnp.int32, (_NUM_CODES, _BB), 0)
    dn = (((1,), (0,)), ((), ()))
    for f in range(_FIELDS):
        r0 = f * (_BATCH // _CHUNK) + _RPB * i
        row = jnp.concatenate(
            [codes_ref[pl.ds(r0 + q, 1), :] for q in range(_RPB)], axis=1)
        bc = jnp.broadcast_to(row, (_NUM_CODES, _BB))
        oht = (bc == cls).astype(jnp.bfloat16)
        acc = lax.dot_general(cbt_ref[...], oht, dn,
                              preferred_element_type=jnp.float32)
        out_ref[f] = acc


@jax.jit
def kernel(input, bit_arr, codebook):
    ids = input.T.reshape(_NW, _NCH, _CHUNK)

    mesh = plsc.VectorSubcoreMesh(core_axis_name="c", subcore_axis_name="s")
    sc_codes = pl.kernel(
        _sc_codes_body,
        out_type=jax.ShapeDtypeStruct((_NW * _NCH, _CHUNK), jnp.int32),
        mesh=mesh,
        scratch_types=[
            pltpu.VMEM((_NCH, _CHUNK), jnp.int32),
            pltpu.VMEM((_NCH, _CHUNK), jnp.int32),
            pltpu.SemaphoreType.DMA,
        ],
        compiler_params=pltpu.CompilerParams(use_tc_tiling_on_sc=False),
    )
    codes = sc_codes(ids, bit_arr)

    cb_hi = codebook.astype(jnp.bfloat16)

    decode = pl.pallas_call(
        _tc_decode_body,
        grid=(_GRID,),
        in_specs=[
            pl.BlockSpec((_NW * _NCH, _CHUNK), lambda i: (0, 0)),
            pl.BlockSpec((_EMBED_DIM, _NUM_CODES), lambda i: (0, 0)),
        ],
        out_specs=pl.BlockSpec((_FIELDS, _EMBED_DIM, _BB), lambda i: (0, 0, i)),
        out_shape=jax.ShapeDtypeStruct((_FIELDS, _EMBED_DIM, _BATCH),
                                       jnp.float32),
    )
    out_t = decode(codes, cb_hi.T)
    return jnp.transpose(out_t, (2, 0, 1))


# speedup vs baseline: 1.2213x; 1.0007x over previous
import jax
import jax.numpy as jnp
from jax import lax
from jax.experimental import pallas as pl
from jax.experimental.pallas import tpu as pltpu
from jax.experimental.pallas import tpu_sc as plsc

_VOCAB = 1000000
_NUM_CODES = 256
_EMBED_DIM = 64
_BATCH = 4096
_FIELDS = 26

_TOTAL = _BATCH * _FIELDS
_NW = 32
_PER_W = _TOTAL // _NW
_CHUNK = 128
_NCH = _PER_W // _CHUNK

_BB = 1024
_GRID = _BATCH // _BB


def _sc_codes_body(ids_hbm, bits_hbm, codes_hbm, idx_v, codes_v, csem):
    wid = lax.axis_index("s") * 2 + lax.axis_index("c")

    pltpu.sync_copy(ids_hbm.at[wid], idx_v)

    def fire(j, carry):
        pltpu.async_copy(bits_hbm.at[idx_v.at[j]], codes_v.at[j], csem)
        return carry
    lax.fori_loop(0, _NCH, fire, None)
    pltpu.make_async_copy(ids_hbm.at[wid], codes_v, csem).wait()

    pltpu.sync_copy(codes_v, codes_hbm.at[pl.ds(wid * _NCH, _NCH)])


_RPB = _BB // _CHUNK


def _tc_decode_body(codes_ref, cbt_ref, out_ref):
    i = pl.program_id(0)
    cls = lax.broadcasted_iota(jnp.int32, (_NUM_CODES, _BB), 0)
    dn = (((1,), (0,)), ((), ()))
    for f in range(_FIELDS):
        r0 = f * (_BATCH // _CHUNK) + _RPB * i
        row = jnp.concatenate(
            [codes_ref[pl.ds(r0 + q, 1), :] for q in range(_RPB)], axis=1)
        bc = jnp.broadcast_to(row, (_NUM_CODES, _BB))
        oht = (bc == cls).astype(jnp.bfloat16)
        acc = lax.dot_general(cbt_ref[...], oht, dn,
                              preferred_element_type=jnp.float32)
        out_ref[f] = acc


@jax.jit
def kernel(input, bit_arr, codebook):
    ids = input.T.reshape(_NW, _NCH, _CHUNK)

    mesh = plsc.VectorSubcoreMesh(core_axis_name="c", subcore_axis_name="s")
    sc_codes = pl.kernel(
        _sc_codes_body,
        out_type=jax.ShapeDtypeStruct((_NW * _NCH, _CHUNK), jnp.int32),
        mesh=mesh,
        scratch_types=[
            pltpu.VMEM((_NCH, _CHUNK), jnp.int32),
            pltpu.VMEM((_NCH, _CHUNK), jnp.int32),
            pltpu.SemaphoreType.DMA,
        ],
        compiler_params=pltpu.CompilerParams(use_tc_tiling_on_sc=False),
    )
    codes = sc_codes(ids, bit_arr)

    cb_hi = codebook.astype(jnp.bfloat16)

    decode = pl.pallas_call(
        _tc_decode_body,
        grid=(_GRID,),
        in_specs=[
            pl.BlockSpec((_NW * _NCH, _CHUNK), lambda i: (0, 0)),
            pl.BlockSpec((_EMBED_DIM, _NUM_CODES), lambda i: (0, 0)),
        ],
        out_specs=pl.BlockSpec((_FIELDS, _EMBED_DIM, _BB), lambda i: (0, 0, i)),
        out_shape=jax.ShapeDtypeStruct((_FIELDS, _EMBED_DIM, _BATCH),
                                       jnp.float32),
        compiler_params=pltpu.CompilerParams(skip_device_barrier=True),
    )
    out_t = decode(codes, cb_hi.T)
    return jnp.transpose(out_t, (2, 0, 1))


# speedup vs baseline: 1.2677x; 1.0380x over previous
import jax
import jax.numpy as jnp
from jax import lax
from jax.experimental import pallas as pl
from jax.experimental.pallas import tpu as pltpu
from jax.experimental.pallas import tpu_sc as plsc

_VOCAB = 1000000
_NUM_CODES = 256
_EMBED_DIM = 64
_BATCH = 4096
_FIELDS = 26

_TOTAL = _BATCH * _FIELDS
_NW = 32
_PER_W = _TOTAL // _NW
_CHUNK = 128
_NCH = _PER_W // _CHUNK

_BB = 1024
_GRID = _BATCH // _BB


def _sc_codes_body(ids_hbm, bits_hbm, codes_hbm, idx_v, codes_v, csem):
    wid = lax.axis_index("s") * 2 + lax.axis_index("c")

    pltpu.sync_copy(ids_hbm.at[wid], idx_v)

    def fire(j, carry):
        pltpu.async_copy(bits_hbm.at[idx_v.at[j]], codes_v.at[j], csem)
        return carry
    lax.fori_loop(0, _NCH, fire, None)
    pltpu.make_async_copy(ids_hbm.at[wid], codes_v, csem).wait()

    pltpu.sync_copy(codes_v, codes_hbm.at[pl.ds(wid * _NCH, _NCH)])


_RPB = _BB // _CHUNK


def _tc_decode_body(codes_ref, cbt_ref, out_ref):
    i = pl.program_id(0)
    cls = lax.broadcasted_iota(jnp.int32, (_NUM_CODES, _BB), 0)
    dn = (((1,), (0,)), ((), ()))
    for f in range(_FIELDS):
        r0 = f * (_BATCH // _CHUNK) + _RPB * i
        row = jnp.concatenate(
            [codes_ref[pl.ds(r0 + q, 1), :] for q in range(_RPB)], axis=1)
        bc = jnp.broadcast_to(row, (_NUM_CODES, _BB))
        oht = (bc == cls).astype(jnp.bfloat16)
        acc2 = lax.dot_general(cbt_ref[...], oht, dn,
                               preferred_element_type=jnp.float32)
        out_ref[f] = acc2[:_EMBED_DIM] + acc2[_EMBED_DIM:]


@jax.jit
def kernel(input, bit_arr, codebook):
    ids = input.T.reshape(_NW, _NCH, _CHUNK)

    mesh = plsc.VectorSubcoreMesh(core_axis_name="c", subcore_axis_name="s")
    sc_codes = pl.kernel(
        _sc_codes_body,
        out_type=jax.ShapeDtypeStruct((_NW * _NCH, _CHUNK), jnp.int32),
        mesh=mesh,
        scratch_types=[
            pltpu.VMEM((_NCH, _CHUNK), jnp.int32),
            pltpu.VMEM((_NCH, _CHUNK), jnp.int32),
            pltpu.SemaphoreType.DMA,
        ],
        compiler_params=pltpu.CompilerParams(use_tc_tiling_on_sc=False),
    )
    codes = sc_codes(ids, bit_arr)

    cb_hi = codebook.astype(jnp.bfloat16)
    cb_lo = (codebook - cb_hi.astype(jnp.float32)).astype(jnp.bfloat16)
    cbt2 = jnp.concatenate([cb_hi.T, cb_lo.T], axis=0)

    decode = pl.pallas_call(
        _tc_decode_body,
        grid=(_GRID,),
        in_specs=[
            pl.BlockSpec((_NW * _NCH, _CHUNK), lambda i: (0, 0)),
            pl.BlockSpec((2 * _EMBED_DIM, _NUM_CODES), lambda i: (0, 0)),
        ],
        out_specs=pl.BlockSpec((_FIELDS, _EMBED_DIM, _BB), lambda i: (0, 0, i)),
        out_shape=jax.ShapeDtypeStruct((_FIELDS, _EMBED_DIM, _BATCH),
                                       jnp.float32),
    )
    out_t = decode(codes, cbt2)
    return jnp.transpose(out_t, (2, 0, 1))
